# Initial kernel scaffold; baseline (speedup 1.0000x reference)
import jax
import jax.numpy as jnp
from jax.experimental import pallas as pl


def kernel(x, edge_index, edge_weight, W1, b1, W2, b2):
    raise NotImplementedError("write your pallas kernel here")


# speedup vs baseline: 2.6359x; 2.6359x over previous
import functools

import jax
import jax.numpy as jnp
from jax import lax
from jax.experimental import pallas as pl
from jax.experimental.pallas import tpu as pltpu
from jax.experimental.pallas import tpu_sc as plsc

N_NODES = 10000
NP = 10240
IN_C, HID_C, OUT_C = 128, 256, 128
BLK = 1024
_INTERPRET = False

NC, NS = 2, 16
W32 = NC * NS
SUB = NP // W32
E = 320000
CH = 800
CB = 1024
B = 128


def _edge_body(C, g_hbm, src_hbm, dst_hbm, ew_hbm, out_hbm,
               src_s, dst_s, ew_s, src_b, dst_b, w_b,
               rows_v, sidx_st, dst_st, w_st, acc_ts):
    c = lax.axis_index("c")
    s = lax.axis_index("s")
    lo = (c * NS + s) * SUB
    nseg = C // 16
    zero16 = jnp.zeros((16,), jnp.float32)
    zi16 = jnp.zeros((16,), jnp.int32)
    col0 = lax.iota(jnp.int32, 16)

    def _zrow(r, _):
        for seg in range(nseg):
            acc_ts[r, pl.ds(seg * 16, 16)] = zero16
        return 0
    lax.fori_loop(0, SUB, _zrow, 0)

    def _fire(fb, _):
        for j in range(8):
            sl = pl.ds(fb * B + j * 16, 16)
            t = pl.ds(j * 16, 16)
            sidx_st[t] = src_b[sl]
            dst_st[t] = dst_b[sl]
            w_st[t] = w_b[sl]
        pltpu.sync_copy(g_hbm.at[sidx_st], rows_v)

        def _row(r, _):
            dr = plsc.load_gather(dst_st, [zi16 + r])
            wr = plsc.load_gather(w_st, [zi16 + r])
            for seg in range(nseg):
                sl2 = pl.ds(seg * 16, 16)
                plsc.addupdate_scatter(acc_ts, [dr, col0 + seg * 16],
                                       rows_v[r, sl2] * wr)
            return 0
        lax.fori_loop(0, B, _row, 0)
        return 0

    def _chunk(ci, T):
        ebase = ci * CH
        pltpu.sync_copy(src_hbm.at[pl.ds(ebase, CH)], src_s)
        pltpu.sync_copy(dst_hbm.at[pl.ds(ebase, CH)], dst_s)
        pltpu.sync_copy(ew_hbm.at[pl.ds(ebase, CH)], ew_s)

        def _grp(gi, off):
            d = dst_s[pl.ds(gi * 16, 16)]
            sv = src_s[pl.ds(gi * 16, 16)]
            w = ew_s[pl.ds(gi * 16, 16)]
            m = (d >= lo) & (d < lo + SUB)
            pos = off + plsc.cumsum(m.astype(jnp.int32)) - 1
            plsc.store_scatter(dst_b, [pos], d - lo, mask=m)
            plsc.store_scatter(src_b, [pos], sv, mask=m)
            plsc.store_scatter(w_b, [pos], w, mask=m)
            return off + jnp.max(plsc.all_reduce_population_count(m))
        T = lax.fori_loop(0, CH // 16, _grp, T)

        nf = lax.shift_right_logical(T, 7)
        lax.fori_loop(0, nf, _fire, 0)
        rb = nf * B
        for j in range(8):
            sl = pl.ds(rb + j * 16, 16)
            t = pl.ds(j * 16, 16)
            src_b[t] = src_b[sl]
            dst_b[t] = dst_b[sl]
            w_b[t] = w_b[sl]
        return T - rb
    T = lax.fori_loop(0, E // CH, _chunk, jnp.int32(0))

    for j in range(8):
        t = pl.ds(T + j * 16, 16)
        dst_b[t] = zi16
        src_b[t] = zi16
        w_b[t] = zero16
    nb = lax.shift_right_logical(T + (B - 1), 7)
    lax.fori_loop(0, nb, _fire, 0)

    pltpu.sync_copy(acc_ts, out_hbm.at[pl.ds(lo, SUB)])


def _deg_body(dst_hbm, ew_hbm, out_hbm, dst_s, ew_s, acc_ts):
    c = lax.axis_index("c")
    s = lax.axis_index("s")
    lo = (c * NS + s) * SUB
    zero16 = jnp.zeros((16,), jnp.float32)
    col0 = lax.iota(jnp.int32, 16)

    def _zrow(r, _):
        acc_ts[r, pl.ds(0, 16)] = zero16
        return 0
    lax.fori_loop(0, SUB, _zrow, 0)

    def _chunk(ci, _):
        ebase = ci * CH
        pltpu.sync_copy(dst_hbm.at[pl.ds(ebase, CH)], dst_s)
        pltpu.sync_copy(ew_hbm.at[pl.ds(ebase, CH)], ew_s)

        def _grp(gi, _):
            d = dst_s[pl.ds(gi * 16, 16)]
            w = ew_s[pl.ds(gi * 16, 16)]
            m = (d >= lo) & (d < lo + SUB)
            dl = jnp.where(m, d - lo, 0)
            plsc.addupdate_scatter(acc_ts, [dl, col0], w, mask=m)
            return 0
        lax.fori_loop(0, CH // 16, _grp, 0)
        return 0
    lax.fori_loop(0, E // CH, _chunk, 0)

    pltpu.sync_copy(acc_ts, out_hbm.at[pl.ds(lo, SUB)])


_deg_kernel = pl.kernel(
    _deg_body,
    out_type=jax.ShapeDtypeStruct((NP, 16), jnp.float32),
    mesh=plsc.VectorSubcoreMesh(core_axis_name="c", subcore_axis_name="s"),
    scratch_types=[
        pltpu.VMEM((CH,), jnp.int32),
        pltpu.VMEM((CH,), jnp.float32),
        pltpu.VMEM((SUB, 16), jnp.float32),
    ],
    name="gcn_deg",
    compiler_params=pltpu.CompilerParams(needs_layout_passes=False),
)


def _make_edge_kernel(C):
    mesh = plsc.VectorSubcoreMesh(core_axis_name="c", subcore_axis_name="s")
    return pl.kernel(
        functools.partial(_edge_body, C),
        out_type=jax.ShapeDtypeStruct((NP, C), jnp.float32),
        mesh=mesh,
        scratch_types=[
            pltpu.VMEM((CH,), jnp.int32),
            pltpu.VMEM((CH,), jnp.int32),
            pltpu.VMEM((CH,), jnp.float32),
            pltpu.VMEM((CB,), jnp.int32),
            pltpu.VMEM((CB,), jnp.int32),
            pltpu.VMEM((CB,), jnp.float32),
            pltpu.VMEM((B, C), jnp.float32),
            pltpu.VMEM((B,), jnp.int32),
            pltpu.VMEM((B,), jnp.int32),
            pltpu.VMEM((B,), jnp.float32),
            pltpu.VMEM((SUB, C), jnp.float32),
        ],
        name=f"gcn_edge_c{C}",
        compiler_params=pltpu.CompilerParams(needs_layout_passes=False),
    )


_edge256 = _make_edge_kernel(HID_C)
_edge128 = _make_edge_kernel(OUT_C)


def _k1_body(deg_ref, x_ref, W_ref, g_ref, dinv_ref):
    deg = jnp.sum(deg_ref[...], axis=1, keepdims=True)
    dinv = jax.lax.rsqrt(1.0 + deg)
    h = jnp.dot(x_ref[...], W_ref[...], preferred_element_type=jnp.float32)
    g_ref[...] = h * dinv
    dinv_ref[...] = dinv


def _k3_body(acc_ref, g_ref, dinv_ref, b_ref, W_ref, g2_ref):
    dinv = dinv_ref[...]
    t = jnp.maximum(dinv * (acc_ref[...] + g_ref[...]) + b_ref[...], 0.0)
    h2 = jnp.dot(t, W_ref[...], preferred_element_type=jnp.float32)
    g2_ref[...] = h2 * dinv


def _k5_body(acc_ref, g_ref, dinv_ref, b_ref, o_ref):
    o = dinv_ref[...] * (acc_ref[...] + g_ref[...]) + b_ref[...]
    m = jnp.max(o, axis=1, keepdims=True)
    e = jnp.exp(o - m)
    o_ref[...] = e / jnp.sum(e, axis=1, keepdims=True)


def _rows(c):
    return pl.BlockSpec((BLK, c), lambda i: (i, 0))


def _full(r, c):
    return pl.BlockSpec((r, c), lambda i: (0, 0))


_k1 = pl.pallas_call(
    _k1_body,
    grid=(NP // BLK,),
    in_specs=[_rows(16), _rows(IN_C), _full(IN_C, HID_C)],
    out_specs=[_rows(HID_C), _rows(1)],
    out_shape=[
        jax.ShapeDtypeStruct((NP, HID_C), jnp.float32),
        jax.ShapeDtypeStruct((NP, 1), jnp.float32),
    ],
    interpret=_INTERPRET,
)

_k3 = pl.pallas_call(
    _k3_body,
    grid=(NP // BLK,),
    in_specs=[_rows(HID_C), _rows(HID_C), _rows(1), _full(1, HID_C),
              _full(HID_C, OUT_C)],
    out_specs=_rows(OUT_C),
    out_shape=jax.ShapeDtypeStruct((NP, OUT_C), jnp.float32),
    interpret=_INTERPRET,
)

_k5 = pl.pallas_call(
    _k5_body,
    grid=(NP // BLK,),
    in_specs=[_rows(OUT_C), _rows(OUT_C), _rows(1), _full(1, OUT_C)],
    out_specs=_rows(OUT_C),
    out_shape=jax.ShapeDtypeStruct((NP, OUT_C), jnp.float32),
    interpret=_INTERPRET,
)

def kernel(x, edge_index, edge_weight, W1, b1, W2, b2):
    src = edge_index[0].astype(jnp.int32)
    dst = edge_index[1].astype(jnp.int32)
    ew = edge_weight

    xp = jnp.zeros((NP, IN_C), jnp.float32).at[:N_NODES].set(x)
    deg16 = _deg_kernel(dst, ew)

    g1, dinv = _k1(deg16, xp, W1)
    acc1 = _edge256(g1, src, dst, ew)
    g2 = _k3(acc1, g1, dinv, b1[None, :], W2)
    acc2 = _edge128(g2, src, dst, ew)
    out = _k5(acc2, g2, dinv, b2[None, :])
    return out[:N_NODES]


# speedup vs baseline: 3.7467x; 1.4214x over previous
import functools

import jax
import jax.numpy as jnp
from jax import lax
from jax.experimental import pallas as pl
from jax.experimental.pallas import tpu as pltpu
from jax.experimental.pallas import tpu_sc as plsc

N_NODES = 10000
NP = 10240
IN_C, HID_C, OUT_C = 128, 256, 128
BLK = 1024
_INTERPRET = False

NC, NS = 2, 16
W32 = NC * NS
SUB = NP // W32
E = 320000


def _edge_body(C, CH, B, g_hbm, src_hbm, dst_hbm, ew_hbm, out_hbm,
               src_s, dst_s, ew_s, src_b, dst_b, w_b,
               rows_v, sidx_st, dst_st, w_st, acc_ts):
    c = lax.axis_index("c")
    s = lax.axis_index("s")
    lo = (c * NS + s) * SUB
    nseg = C // 16
    nbg = B // 16
    log2b = B.bit_length() - 1
    zero16 = jnp.zeros((16,), jnp.float32)
    zi16 = jnp.zeros((16,), jnp.int32)
    col0 = lax.iota(jnp.int32, 16)

    def _zrow(r, _):
        for seg in range(nseg):
            acc_ts[r, pl.ds(seg * 16, 16)] = zero16
        return 0
    lax.fori_loop(0, SUB, _zrow, 0)

    def _fire(fb, _):
        for j in range(nbg):
            sl = pl.ds(fb * B + j * 16, 16)
            t = pl.ds(j * 16, 16)
            sidx_st[t] = src_b[sl]
            dst_st[t] = dst_b[sl]
            w_st[t] = w_b[sl]
        pltpu.sync_copy(g_hbm.at[sidx_st], rows_v)

        def _row(r, _):
            dr = plsc.load_gather(dst_st, [zi16 + r])
            wr = plsc.load_gather(w_st, [zi16 + r])
            for seg in range(nseg):
                sl2 = pl.ds(seg * 16, 16)
                plsc.addupdate_scatter(acc_ts, [dr, col0 + seg * 16],
                                       rows_v[r, sl2] * wr)
            return 0
        lax.fori_loop(0, B, _row, 0)
        return 0

    def _chunk(ci, T):
        ebase = ci * CH
        pltpu.sync_copy(src_hbm.at[pl.ds(ebase, CH)], src_s)
        pltpu.sync_copy(dst_hbm.at[pl.ds(ebase, CH)], dst_s)
        pltpu.sync_copy(ew_hbm.at[pl.ds(ebase, CH)], ew_s)

        def _grp(gi, off):
            d = dst_s[pl.ds(gi * 16, 16)]
            sv = src_s[pl.ds(gi * 16, 16)]
            w = ew_s[pl.ds(gi * 16, 16)]
            m = (d >= lo) & (d < lo + SUB)
            pos = off + plsc.cumsum(m.astype(jnp.int32)) - 1
            plsc.store_scatter(dst_b, [pos], d - lo, mask=m)
            plsc.store_scatter(src_b, [pos], sv, mask=m)
            plsc.store_scatter(w_b, [pos], w, mask=m)
            return off + jnp.max(plsc.all_reduce_population_count(m))
        T = lax.fori_loop(0, CH // 16, _grp, T)

        nf = lax.shift_right_logical(T, log2b)
        lax.fori_loop(0, nf, _fire, 0)
        rb = nf * B
        for j in range(nbg):
            sl = pl.ds(rb + j * 16, 16)
            t = pl.ds(j * 16, 16)
            src_b[t] = src_b[sl]
            dst_b[t] = dst_b[sl]
            w_b[t] = w_b[sl]
        return T - rb
    T = lax.fori_loop(0, E // CH, _chunk, jnp.int32(0))

    for j in range(nbg):
        t = pl.ds(T + j * 16, 16)
        dst_b[t] = zi16
        src_b[t] = zi16
        w_b[t] = zero16
    nb = lax.shift_right_logical(T + (B - 1), log2b)
    lax.fori_loop(0, nb, _fire, 0)

    pltpu.sync_copy(acc_ts, out_hbm.at[pl.ds(lo, SUB)])


def _deg_body(CH, dst_hbm, ew_hbm, out_hbm, dst_s, ew_s, acc_ts):
    c = lax.axis_index("c")
    s = lax.axis_index("s")
    lo = (c * NS + s) * SUB
    zero16 = jnp.zeros((16,), jnp.float32)
    col0 = lax.iota(jnp.int32, 16)

    def _zrow(r, _):
        acc_ts[r, pl.ds(0, 16)] = zero16
        return 0
    lax.fori_loop(0, SUB, _zrow, 0)

    def _chunk(ci, _):
        ebase = ci * CH
        pltpu.sync_copy(dst_hbm.at[pl.ds(ebase, CH)], dst_s)
        pltpu.sync_copy(ew_hbm.at[pl.ds(ebase, CH)], ew_s)

        def _grp(gi, _):
            d = dst_s[pl.ds(gi * 16, 16)]
            w = ew_s[pl.ds(gi * 16, 16)]
            m = (d >= lo) & (d < lo + SUB)
            dl = jnp.where(m, d - lo, 0)
            plsc.addupdate_scatter(acc_ts, [dl, col0], w, mask=m)
            return 0
        lax.fori_loop(0, CH // 16, _grp, 0)
        return 0
    lax.fori_loop(0, E // CH, _chunk, 0)

    pltpu.sync_copy(acc_ts, out_hbm.at[pl.ds(lo, SUB)])


DEG_CH = 16000

_deg_kernel = pl.kernel(
    functools.partial(_deg_body, DEG_CH),
    out_type=jax.ShapeDtypeStruct((NP, 16), jnp.float32),
    mesh=plsc.VectorSubcoreMesh(core_axis_name="c", subcore_axis_name="s"),
    scratch_types=[
        pltpu.VMEM((DEG_CH,), jnp.int32),
        pltpu.VMEM((DEG_CH,), jnp.float32),
        pltpu.VMEM((SUB, 16), jnp.float32),
    ],
    name="gcn_deg",
    compiler_params=pltpu.CompilerParams(needs_layout_passes=False),
)


def _make_edge_kernel(C, CH, B):
    CB = CH + B
    mesh = plsc.VectorSubcoreMesh(core_axis_name="c", subcore_axis_name="s")
    return pl.kernel(
        functools.partial(_edge_body, C, CH, B),
        out_type=jax.ShapeDtypeStruct((NP, C), jnp.float32),
        mesh=mesh,
        scratch_types=[
            pltpu.VMEM((CH,), jnp.int32),
            pltpu.VMEM((CH,), jnp.int32),
            pltpu.VMEM((CH,), jnp.float32),
            pltpu.VMEM((CB,), jnp.int32),
            pltpu.VMEM((CB,), jnp.int32),
            pltpu.VMEM((CB,), jnp.float32),
            pltpu.VMEM((B, C), jnp.float32),
            pltpu.VMEM((B,), jnp.int32),
            pltpu.VMEM((B,), jnp.int32),
            pltpu.VMEM((B,), jnp.float32),
            pltpu.VMEM((SUB, C), jnp.float32),
        ],
        name=f"gcn_edge_c{C}",
        compiler_params=pltpu.CompilerParams(needs_layout_passes=False),
    )


_edge256 = _make_edge_kernel(HID_C, 2000, 128)
_edge128 = _make_edge_kernel(OUT_C, 6400, 256)


def _k1_body(deg_ref, x_ref, W_ref, g_ref, dinv_ref):
    deg = jnp.sum(deg_ref[...], axis=1, keepdims=True)
    dinv = jax.lax.rsqrt(1.0 + deg)
    h = jnp.dot(x_ref[...], W_ref[...], preferred_element_type=jnp.float32)
    g_ref[...] = h * dinv
    dinv_ref[...] = dinv


def _k3_body(acc_ref, g_ref, dinv_ref, b_ref, W_ref, g2_ref):
    dinv = dinv_ref[...]
    t = jnp.maximum(dinv * (acc_ref[...] + g_ref[...]) + b_ref[...], 0.0)
    h2 = jnp.dot(t, W_ref[...], preferred_element_type=jnp.float32)
    g2_ref[...] = h2 * dinv


def _k5_body(acc_ref, g_ref, dinv_ref, b_ref, o_ref):
    o = dinv_ref[...] * (acc_ref[...] + g_ref[...]) + b_ref[...]
    m = jnp.max(o, axis=1, keepdims=True)
    e = jnp.exp(o - m)
    o_ref[...] = e / jnp.sum(e, axis=1, keepdims=True)


def _rows(c):
    return pl.BlockSpec((BLK, c), lambda i: (i, 0))


def _full(r, c):
    return pl.BlockSpec((r, c), lambda i: (0, 0))


_k1 = pl.pallas_call(
    _k1_body,
    grid=(NP // BLK,),
    in_specs=[_rows(16), _rows(IN_C), _full(IN_C, HID_C)],
    out_specs=[_rows(HID_C), _rows(1)],
    out_shape=[
        jax.ShapeDtypeStruct((NP, HID_C), jnp.float32),
        jax.ShapeDtypeStruct((NP, 1), jnp.float32),
    ],
    interpret=_INTERPRET,
)

_k3 = pl.pallas_call(
    _k3_body,
    grid=(NP // BLK,),
    in_specs=[_rows(HID_C), _rows(HID_C), _rows(1), _full(1, HID_C),
              _full(HID_C, OUT_C)],
    out_specs=_rows(OUT_C),
    out_shape=jax.ShapeDtypeStruct((NP, OUT_C), jnp.float32),
    interpret=_INTERPRET,
)

_k5 = pl.pallas_call(
    _k5_body,
    grid=(NP // BLK,),
    in_specs=[_rows(OUT_C), _rows(OUT_C), _rows(1), _full(1, OUT_C)],
    out_specs=_rows(OUT_C),
    out_shape=jax.ShapeDtypeStruct((NP, OUT_C), jnp.float32),
    interpret=_INTERPRET,
)

def kernel(x, edge_index, edge_weight, W1, b1, W2, b2):
    src = edge_index[0].astype(jnp.int32)
    dst = edge_index[1].astype(jnp.int32)
    ew = edge_weight

    xp = jnp.zeros((NP, IN_C), jnp.float32).at[:N_NODES].set(x)
    deg16 = _deg_kernel(dst, ew)

    g1, dinv = _k1(deg16, xp, W1)
    acc1 = _edge256(g1, src, dst, ew)
    g2 = _k3(acc1, g1, dinv, b1[None, :], W2)
    acc2 = _edge128(g2, src, dst, ew)
    out = _k5(acc2, g2, dinv, b2[None, :])
    return out[:N_NODES]


# speedup vs baseline: 4.7925x; 1.2791x over previous
import functools

import jax
import jax.numpy as jnp
from jax import lax
from jax.experimental import pallas as pl
from jax.experimental.pallas import tpu as pltpu
from jax.experimental.pallas import tpu_sc as plsc

N_NODES = 10000
NP = 10240
IN_C, HID_C, OUT_C = 128, 256, 128
BLK = 1024
_INTERPRET = False

NC, NS = 2, 16
W32 = NC * NS
SUB = NP // W32
E = 320000
PCH = 8000
F = 512
LOG2F = 9
PCB = PCH + 2 * F
EMAX = E + F


def _prep_body(src_hbm, dst_hbm, ew_hbm,
               deg_hbm, csrc_hbm, cdst_hbm, cw_hbm, cnt_hbm,
               src_s, dst_s, ew_s, src_b, dst_b, w_b, acc_ts, cnt_v):
    c = lax.axis_index("c")
    s = lax.axis_index("s")
    row = c * NS + s
    lo = row * SUB
    base = row * EMAX
    zero16 = jnp.zeros((16,), jnp.float32)
    zi16 = jnp.zeros((16,), jnp.int32)
    col0 = lax.iota(jnp.int32, 16)

    def _zrow(r, _):
        acc_ts[r, pl.ds(0, 16)] = zero16
        return 0
    lax.fori_loop(0, SUB, _zrow, 0)

    def _chunk(ci, TW):
        T, Wb = TW
        ebase = ci * PCH
        pltpu.sync_copy(src_hbm.at[pl.ds(ebase, PCH)], src_s)
        pltpu.sync_copy(dst_hbm.at[pl.ds(ebase, PCH)], dst_s)
        pltpu.sync_copy(ew_hbm.at[pl.ds(ebase, PCH)], ew_s)

        def _grp(gi, T):
            d = dst_s[pl.ds(gi * 16, 16)]
            sv = src_s[pl.ds(gi * 16, 16)]
            w = ew_s[pl.ds(gi * 16, 16)]
            m = (d >= lo) & (d < lo + SUB)
            dl = jnp.where(m, d - lo, 0)
            plsc.addupdate_scatter(acc_ts, [dl, col0], w, mask=m)
            pos = T + plsc.cumsum(m.astype(jnp.int32)) - 1
            plsc.store_scatter(dst_b, [pos], dl, mask=m)
            plsc.store_scatter(src_b, [pos], sv, mask=m)
            plsc.store_scatter(w_b, [pos], w, mask=m)
            return T + jnp.max(plsc.all_reduce_population_count(m))
        T = lax.fori_loop(0, PCH // 16, _grp, T)

        nf = lax.shift_right_logical(T, LOG2F)

        def _flush(fb, _):
            sl = pl.ds(fb * F, F)
            ob = pl.ds(base + (Wb + fb) * F, F)
            pltpu.sync_copy(src_b.at[sl], csrc_hbm.at[ob])
            pltpu.sync_copy(dst_b.at[sl], cdst_hbm.at[ob])
            pltpu.sync_copy(w_b.at[sl], cw_hbm.at[ob])
            return 0
        lax.fori_loop(0, nf, _flush, 0)
        rb = nf * F
        for j in range(F // 16):
            sl = pl.ds(rb + j * 16, 16)
            t = pl.ds(j * 16, 16)
            src_b[t] = src_b[sl]
            dst_b[t] = dst_b[sl]
            w_b[t] = w_b[sl]
        return (T - rb, Wb + nf)
    T, Wb = lax.fori_loop(0, E // PCH, _chunk,
                          (jnp.int32(0), jnp.int32(0)))

    for j in range(F // 16):
        t = pl.ds(j * 16, 16)
        m = (j * 16 + col0) < T
        src_b[t] = jnp.where(m, src_b[t], 0)
        dst_b[t] = jnp.where(m, dst_b[t], 0)
        w_b[t] = jnp.where(m, w_b[t], 0.0)
    sl = pl.ds(0, F)
    ob = pl.ds(base + Wb * F, F)
    pltpu.sync_copy(src_b.at[sl], csrc_hbm.at[ob])
    pltpu.sync_copy(dst_b.at[sl], cdst_hbm.at[ob])
    pltpu.sync_copy(w_b.at[sl], cw_hbm.at[ob])

    cnt_v[pl.ds(0, 16)] = zi16 + (Wb * F + T)
    pltpu.sync_copy(cnt_v, cnt_hbm.at[pl.ds(row * 16, 16)])
    pltpu.sync_copy(acc_ts, deg_hbm.at[pl.ds(lo, SUB)])


_prep_kernel = pl.kernel(
    _prep_body,
    out_type=[
        jax.ShapeDtypeStruct((NP, 16), jnp.float32),
        jax.ShapeDtypeStruct((W32 * EMAX,), jnp.int32),
        jax.ShapeDtypeStruct((W32 * EMAX,), jnp.int32),
        jax.ShapeDtypeStruct((W32 * EMAX,), jnp.float32),
        jax.ShapeDtypeStruct((W32 * 16,), jnp.int32),
    ],
    mesh=plsc.VectorSubcoreMesh(core_axis_name="c", subcore_axis_name="s"),
    scratch_types=[
        pltpu.VMEM((PCH,), jnp.int32),
        pltpu.VMEM((PCH,), jnp.int32),
        pltpu.VMEM((PCH,), jnp.float32),
        pltpu.VMEM((PCB,), jnp.int32),
        pltpu.VMEM((PCB,), jnp.int32),
        pltpu.VMEM((PCB,), jnp.float32),
        pltpu.VMEM((SUB, 16), jnp.float32),
        pltpu.VMEM((16,), jnp.int32),
    ],
    name="gcn_prep",
    compiler_params=pltpu.CompilerParams(needs_layout_passes=False),
)


def _edge_body(C, B, g_hbm, csrc_hbm, cdst_hbm, cw_hbm, cnt_hbm, out_hbm,
               rows_v, sidx_st, dst_st, w_st, cnt_v, acc_ts):
    c = lax.axis_index("c")
    s = lax.axis_index("s")
    row = c * NS + s
    lo = row * SUB
    base = row * EMAX
    nseg = C // 16
    log2b = B.bit_length() - 1
    zero16 = jnp.zeros((16,), jnp.float32)
    zi16 = jnp.zeros((16,), jnp.int32)
    col0 = lax.iota(jnp.int32, 16)

    def _zrow(r, _):
        for seg in range(nseg):
            acc_ts[r, pl.ds(seg * 16, 16)] = zero16
        return 0
    lax.fori_loop(0, SUB, _zrow, 0)

    pltpu.sync_copy(cnt_hbm.at[pl.ds(row * 16, 16)], cnt_v)
    n = jnp.max(cnt_v[pl.ds(0, 16)])
    nb = lax.shift_right_logical(n + (B - 1), log2b)

    def _fire(fb, _):
        eb = pl.ds(base + fb * B, B)
        pltpu.sync_copy(csrc_hbm.at[eb], sidx_st)
        pltpu.sync_copy(cdst_hbm.at[eb], dst_st)
        pltpu.sync_copy(cw_hbm.at[eb], w_st)
        pltpu.sync_copy(g_hbm.at[sidx_st], rows_v)

        def _row(r, _):
            dr = plsc.load_gather(dst_st, [zi16 + r])
            wr = plsc.load_gather(w_st, [zi16 + r])
            for seg in range(nseg):
                sl2 = pl.ds(seg * 16, 16)
                plsc.addupdate_scatter(acc_ts, [dr, col0 + seg * 16],
                                       rows_v[r, sl2] * wr)
            return 0
        lax.fori_loop(0, B, _row, 0)
        return 0
    lax.fori_loop(0, nb, _fire, 0)

    pltpu.sync_copy(acc_ts, out_hbm.at[pl.ds(lo, SUB)])


def _make_edge_kernel(C, B):
    mesh = plsc.VectorSubcoreMesh(core_axis_name="c", subcore_axis_name="s")
    return pl.kernel(
        functools.partial(_edge_body, C, B),
        out_type=jax.ShapeDtypeStruct((NP, C), jnp.float32),
        mesh=mesh,
        scratch_types=[
            pltpu.VMEM((B, C), jnp.float32),
            pltpu.VMEM((B,), jnp.int32),
            pltpu.VMEM((B,), jnp.int32),
            pltpu.VMEM((B,), jnp.float32),
            pltpu.VMEM((16,), jnp.int32),
            pltpu.VMEM((SUB, C), jnp.float32),
        ],
        name=f"gcn_edge_c{C}",
        compiler_params=pltpu.CompilerParams(needs_layout_passes=False),
    )


_edge256 = _make_edge_kernel(HID_C, 128)
_edge128 = _make_edge_kernel(OUT_C, 256)


def _k1_body(deg_ref, x_ref, W_ref, g_ref, dinv_ref):
    deg = jnp.sum(deg_ref[...], axis=1, keepdims=True)
    dinv = jax.lax.rsqrt(1.0 + deg)
    h = jnp.dot(x_ref[...], W_ref[...], preferred_element_type=jnp.float32)
    g_ref[...] = h * dinv
    dinv_ref[...] = dinv


def _k3_body(acc_ref, g_ref, dinv_ref, b_ref, W_ref, g2_ref):
    dinv = dinv_ref[...]
    t = jnp.maximum(dinv * (acc_ref[...] + g_ref[...]) + b_ref[...], 0.0)
    h2 = jnp.dot(t, W_ref[...], preferred_element_type=jnp.float32)
    g2_ref[...] = h2 * dinv


def _k5_body(acc_ref, g_ref, dinv_ref, b_ref, o_ref):
    o = dinv_ref[...] * (acc_ref[...] + g_ref[...]) + b_ref[...]
    m = jnp.max(o, axis=1, keepdims=True)
    e = jnp.exp(o - m)
    o_ref[...] = e / jnp.sum(e, axis=1, keepdims=True)


def _rows(c):
    return pl.BlockSpec((BLK, c), lambda i: (i, 0))


def _full(r, c):
    return pl.BlockSpec((r, c), lambda i: (0, 0))


_k1 = pl.pallas_call(
    _k1_body,
    grid=(NP // BLK,),
    in_specs=[_rows(16), _rows(IN_C), _full(IN_C, HID_C)],
    out_specs=[_rows(HID_C), _rows(1)],
    out_shape=[
        jax.ShapeDtypeStruct((NP, HID_C), jnp.float32),
        jax.ShapeDtypeStruct((NP, 1), jnp.float32),
    ],
    interpret=_INTERPRET,
)

_k3 = pl.pallas_call(
    _k3_body,
    grid=(NP // BLK,),
    in_specs=[_rows(HID_C), _rows(HID_C), _rows(1), _full(1, HID_C),
              _full(HID_C, OUT_C)],
    out_specs=_rows(OUT_C),
    out_shape=jax.ShapeDtypeStruct((NP, OUT_C), jnp.float32),
    interpret=_INTERPRET,
)

_k5 = pl.pallas_call(
    _k5_body,
    grid=(NP // BLK,),
    in_specs=[_rows(OUT_C), _rows(OUT_C), _rows(1), _full(1, OUT_C)],
    out_specs=_rows(OUT_C),
    out_shape=jax.ShapeDtypeStruct((NP, OUT_C), jnp.float32),
    interpret=_INTERPRET,
)

def kernel(x, edge_index, edge_weight, W1, b1, W2, b2):
    src = edge_index[0].astype(jnp.int32)
    dst = edge_index[1].astype(jnp.int32)
    ew = edge_weight

    xp = jnp.zeros((NP, IN_C), jnp.float32).at[:N_NODES].set(x)
    deg16, csrc, cdst, cw, cnt = _prep_kernel(src, dst, ew)

    g1, dinv = _k1(deg16, xp, W1)
    acc1 = _edge256(g1, csrc, cdst, cw, cnt)
    g2 = _k3(acc1, g1, dinv, b1[None, :], W2)
    acc2 = _edge128(g2, csrc, cdst, cw, cnt)
    out = _k5(acc2, g2, dinv, b2[None, :])
    return out[:N_NODES]
